# baseline (speedup 1.0000x reference)
import functools

import jax
import jax.numpy as jnp
from jax import lax
from jax.experimental import pallas as pl
from jax.experimental.pallas import tpu as pltpu
from jax.experimental.pallas import tpu_sc as plsc

NUM_TILES = 32
CHUNK = 128


def _precompute(node_attr, wsT, wrT, block=2000):
    n, nd = node_attr.shape
    grid = n // block

    def body(x_ref, ws_ref, wr_ref, ps_ref, pr_ref):
        x = x_ref[...]
        ps_ref[...] = jnp.dot(
            x, ws_ref[...], preferred_element_type=jnp.float32
        ).astype(jnp.bfloat16)
        pr_ref[...] = jnp.dot(
            x, wr_ref[...], preferred_element_type=jnp.float32
        ).astype(jnp.bfloat16)

    return pl.pallas_call(
        body,
        grid=(grid,),
        in_specs=[
            pl.BlockSpec((block, nd), lambda i: (i, 0)),
            pl.BlockSpec(wsT.shape, lambda i: (0, 0)),
            pl.BlockSpec(wrT.shape, lambda i: (0, 0)),
        ],
        out_specs=[
            pl.BlockSpec((block, wsT.shape[1]), lambda i: (i, 0)),
            pl.BlockSpec((block, wrT.shape[1]), lambda i: (i, 0)),
        ],
        out_shape=[
            jax.ShapeDtypeStruct((n, wsT.shape[1]), jnp.bfloat16),
            jax.ShapeDtypeStruct((n, wrT.shape[1]), jnp.bfloat16),
        ],
    )(node_attr, wsT, wrT)


def _sc_gather(ps, pr, row, col):
    e = row.shape[0]
    d = ps.shape[1]
    per_tile = e // NUM_TILES
    nfull = per_tile // CHUNK
    tail = per_tile - nfull * CHUNK

    mesh = plsc.VectorSubcoreMesh(core_axis_name="c", subcore_axis_name="s")

    @functools.partial(
        pl.kernel,
        mesh=mesh,
        out_type=(
            jax.ShapeDtypeStruct((e, d), jnp.float32),
            jax.ShapeDtypeStruct((e, d), jnp.float32),
        ),
        scratch_types=[
            pltpu.VMEM((CHUNK,), jnp.int32),
            pltpu.VMEM((CHUNK,), jnp.int32),
            pltpu.VMEM((CHUNK, d), jnp.float32),
            pltpu.VMEM((CHUNK, d), jnp.float32),
            pltpu.VMEM((tail,), jnp.int32),
            pltpu.VMEM((tail,), jnp.int32),
            pltpu.VMEM((tail, d), jnp.float32),
            pltpu.VMEM((tail, d), jnp.float32),
            pltpu.SemaphoreType.DMA,
            pltpu.SemaphoreType.DMA,
        ],
        compiler_params=pltpu.CompilerParams(use_tc_tiling_on_sc=False),
    )
    def k(ps_hbm, pr_hbm, row_hbm, col_hbm, s1_hbm, s2_hbm,
          i1, i2, r1, r2, ti1, ti2, tr1, tr2, m1, m2):
        wid = lax.axis_index("c") * 16 + lax.axis_index("s")
        tbase = wid * per_tile

        def body(t, carry):
            base = tbase + t * CHUNK
            pltpu.sync_copy(row_hbm.at[pl.ds(base, CHUNK)], i1)
            pltpu.sync_copy(col_hbm.at[pl.ds(base, CHUNK)], i2)
            cp1 = pltpu.async_copy(ps_hbm.at[i1], r1, m1)
            cp2 = pltpu.async_copy(pr_hbm.at[i2], r2, m2)
            cp1.wait()
            cp2.wait()
            pltpu.sync_copy(r1, s1_hbm.at[pl.ds(base, CHUNK)])
            pltpu.sync_copy(r2, s2_hbm.at[pl.ds(base, CHUNK)])
            return carry

        lax.fori_loop(0, nfull, body, 0)

        if tail:
            base = tbase + nfull * CHUNK
            pltpu.sync_copy(row_hbm.at[pl.ds(base, tail)], ti1)
            pltpu.sync_copy(col_hbm.at[pl.ds(base, tail)], ti2)
            cp1 = pltpu.async_copy(ps_hbm.at[ti1], tr1, m1)
            cp2 = pltpu.async_copy(pr_hbm.at[ti2], tr2, m2)
            cp1.wait()
            cp2.wait()
            pltpu.sync_copy(tr1, s1_hbm.at[pl.ds(base, tail)])
            pltpu.sync_copy(tr2, s2_hbm.at[pl.ds(base, tail)])

    return k(ps, pr, row, col)


def _edge_mlp(s1, s2, ea, weT, b1, w2T, b2, w3T, b3, g, bb, block=2560):
    e, h = s1.shape
    ed = ea.shape[1]
    grid = e // block

    def body(s1_ref, s2_ref, ea_ref, we_ref, b1_ref, w2_ref, b2_ref,
             w3_ref, b3_ref, g_ref, bb_ref, out_ref):
        ea_blk = ea_ref[...]
        s12 = s1_ref[...].astype(jnp.float32) + s2_ref[...].astype(jnp.float32)
        x = (jnp.dot(ea_blk, we_ref[...], preferred_element_type=jnp.float32)
             + s12 + b1_ref[...])
        h1 = jnp.maximum(x, 0.0)
        h2 = jnp.maximum(
            jnp.dot(h1, w2_ref[...], preferred_element_type=jnp.float32)
            + b2_ref[...], 0.0)
        u = jnp.dot(h2, w3_ref[...], preferred_element_type=jnp.float32) + b3_ref[...]
        m = jnp.mean(u, axis=-1, keepdims=True)
        c = u - m
        v = jnp.mean(c * c, axis=-1, keepdims=True)
        ln = c * lax.rsqrt(v + 1e-5) * g_ref[...] + bb_ref[...]
        out_ref[...] = ea_blk + ln

    return pl.pallas_call(
        body,
        grid=(grid,),
        in_specs=[
            pl.BlockSpec((block, h), lambda i: (i, 0)),
            pl.BlockSpec((block, h), lambda i: (i, 0)),
            pl.BlockSpec((block, ed), lambda i: (i, 0)),
            pl.BlockSpec(weT.shape, lambda i: (0, 0)),
            pl.BlockSpec(b1.shape, lambda i: (0, 0)),
            pl.BlockSpec(w2T.shape, lambda i: (0, 0)),
            pl.BlockSpec(b2.shape, lambda i: (0, 0)),
            pl.BlockSpec(w3T.shape, lambda i: (0, 0)),
            pl.BlockSpec(b3.shape, lambda i: (0, 0)),
            pl.BlockSpec(g.shape, lambda i: (0, 0)),
            pl.BlockSpec(bb.shape, lambda i: (0, 0)),
        ],
        out_specs=pl.BlockSpec((block, ed), lambda i: (i, 0)),
        out_shape=jax.ShapeDtypeStruct((e, ed), jnp.float32),
    )(s1, s2, ea, weT, b1, w2T, b2, w3T, b3, g, bb)


def _sc_scatter(col, edges_flat, n_pad, d):
    e = edges_flat.shape[0] // d
    cs = 256
    total_chunks = e // cs
    base_chunks = total_chunks // NUM_TILES
    rem_chunks = total_chunks % NUM_TILES
    half = n_pad // 2
    acc_w = half * d
    acc_pad = acc_w + d
    lanes = 16

    mesh = plsc.VectorSubcoreMesh(core_axis_name="c", subcore_axis_name="s")

    @functools.partial(
        pl.kernel,
        mesh=mesh,
        out_type=jax.ShapeDtypeStruct((NUM_TILES * 2 * acc_w,), jnp.float32),
        scratch_types=[
            pltpu.VMEM((cs + 16,), jnp.int32),
            pltpu.VMEM((cs * 16,), jnp.float32),
            pltpu.VMEM((acc_pad,), jnp.float32),
        ],
    )
    def k(col_hbm, edges_hbm, out_hbm, vcol, rows, acc):
        cid = lax.axis_index("c")
        sid = lax.axis_index("s")
        wid = cid * 16 + sid
        my_chunks = base_chunks + jnp.where(wid < rem_chunks, 1, 0)
        start_chunk = wid * base_chunks + jnp.minimum(wid, rem_chunks)
        zvec = jnp.zeros((lanes,), jnp.float32)

        def do_pass(p):
            lo = p * half

            def zbody(i, carry):
                acc[pl.ds(i * lanes, lanes)] = zvec
                return carry
            lax.fori_loop(0, acc_pad // lanes, zbody, 0, unroll=8)

            def body(t, carry):
                base = (start_chunk + t) * cs
                pltpu.sync_copy(col_hbm.at[pl.ds(base, cs)],
                                vcol.at[pl.ds(0, cs)])
                pltpu.sync_copy(edges_hbm.at[pl.ds(base * d, cs * d)],
                                rows)

                def ebody(e2, carry2):
                    loc = vcol[pl.ds(e2, lanes)][0] - lo
                    inr = (loc >= 0) & (loc < half)
                    off = jnp.where(inr, loc, half) * d
                    acc[pl.ds(off, lanes)] = (
                        acc[pl.ds(off, lanes)] + rows[pl.ds(e2 * d, lanes)])
                    return carry2
                lax.fori_loop(0, cs, ebody, 0, unroll=8)
                return carry
            lax.fori_loop(0, my_chunks, body, 0)

            pltpu.sync_copy(
                acc.at[pl.ds(0, acc_w)],
                out_hbm.at[pl.ds((wid * 2 + p) * acc_w, acc_w)])

        do_pass(0)
        do_pass(1)

    return k(col, edges_flat)


def _sum_partials(parts, block=2048):
    nparts, n_pad, ed = parts.shape
    grid_n = n_pad // block

    def body(p_ref, out_ref):
        k = pl.program_id(1)

        @pl.when(k == 0)
        def _():
            out_ref[...] = jnp.zeros_like(out_ref)

        out_ref[...] += p_ref[0]

    return pl.pallas_call(
        body,
        grid=(grid_n, nparts),
        in_specs=[pl.BlockSpec((1, block, ed), lambda i, k: (k, i, 0))],
        out_specs=pl.BlockSpec((block, ed), lambda i, k: (i, 0)),
        out_shape=jax.ShapeDtypeStruct((n_pad, ed), jnp.float32),
    )(parts)


def _node_mlp(na, aggr_in, w1aT, w1bT, b1, w2T, b2, w3T, b3, g, bb, block=2000):
    n, nd = na.shape
    ed = aggr_in.shape[1]
    grid = n // block

    def body(na_ref, aggr_ref, w1a_ref, w1b_ref, b1_ref, w2_ref,
             b2_ref, w3_ref, b3_ref, g_ref, bb_ref, out_ref):
        na_blk = na_ref[...]
        aggr = aggr_ref[...]
        x = (jnp.dot(na_blk, w1a_ref[...], preferred_element_type=jnp.float32)
             + jnp.dot(aggr, w1b_ref[...], preferred_element_type=jnp.float32)
             + b1_ref[...])
        h1 = jnp.maximum(x, 0.0)
        h2 = jnp.maximum(
            jnp.dot(h1, w2_ref[...], preferred_element_type=jnp.float32)
            + b2_ref[...], 0.0)
        u = jnp.dot(h2, w3_ref[...], preferred_element_type=jnp.float32) + b3_ref[...]
        m = jnp.mean(u, axis=-1, keepdims=True)
        c = u - m
        v = jnp.mean(c * c, axis=-1, keepdims=True)
        ln = c * lax.rsqrt(v + 1e-5) * g_ref[...] + bb_ref[...]
        out_ref[...] = na_blk + ln

    return pl.pallas_call(
        body,
        grid=(grid,),
        in_specs=[
            pl.BlockSpec((block, nd), lambda i: (i, 0)),
            pl.BlockSpec((block, ed), lambda i: (i, 0)),
            pl.BlockSpec(w1aT.shape, lambda i: (0, 0)),
            pl.BlockSpec(w1bT.shape, lambda i: (0, 0)),
            pl.BlockSpec(b1.shape, lambda i: (0, 0)),
            pl.BlockSpec(w2T.shape, lambda i: (0, 0)),
            pl.BlockSpec(b2.shape, lambda i: (0, 0)),
            pl.BlockSpec(w3T.shape, lambda i: (0, 0)),
            pl.BlockSpec(b3.shape, lambda i: (0, 0)),
            pl.BlockSpec(g.shape, lambda i: (0, 0)),
            pl.BlockSpec(bb.shape, lambda i: (0, 0)),
        ],
        out_specs=pl.BlockSpec((block, nd), lambda i: (i, 0)),
        out_shape=jax.ShapeDtypeStruct((n, nd), jnp.float32),
    )(na, aggr_in, w1aT, w1bT, b1, w2T, b2, w3T, b3, g, bb)


def kernel(node_attr, edge_attr, edge_index,
           eW1, eb1, eW2, eb2, eW3, eb3, eLNg, eLNb,
           nW1, nb1, nW2, nb2, nW3, nb3, nLNg, nLNb):
    n, nd = node_attr.shape
    e, ed = edge_attr.shape

    row = edge_index[0]
    col = edge_index[1]

    weT = eW1[:, :ed].T
    wsT = eW1[:, ed:ed + nd].T
    wrT = eW1[:, ed + nd:].T
    ew2T = eW2.T
    ew3T = eW3.T
    eb1r = eb1.reshape(1, -1)
    eb2r = eb2.reshape(1, -1)
    eb3r = eb3.reshape(1, -1)
    eg = eLNg.reshape(1, -1)
    ebb = eLNb.reshape(1, -1)

    nw1aT = nW1[:, :nd].T
    nw1bT = nW1[:, nd:].T
    nw2T = nW2.T
    nw3T = nW3.T
    nb1r = nb1.reshape(1, -1)
    nb2r = nb2.reshape(1, -1)
    nb3r = nb3.reshape(1, -1)
    ng = nLNg.reshape(1, -1)
    nbb = nLNb.reshape(1, -1)

    ps, pr = _precompute(node_attr, wsT, wrT)
    ps32 = lax.bitcast_convert_type(ps.reshape(n, nd // 2, 2), jnp.float32)
    pr32 = lax.bitcast_convert_type(pr.reshape(n, nd // 2, 2), jnp.float32)
    s1w, s2w = _sc_gather(ps32, pr32, row, col)
    s1 = lax.bitcast_convert_type(s1w, jnp.bfloat16).reshape(e, nd)
    s2 = lax.bitcast_convert_type(s2w, jnp.bfloat16).reshape(e, nd)
    edge_attr_new = _edge_mlp(s1, s2, edge_attr, weT, eb1r,
                              ew2T, eb2r, ew3T, eb3r, eg, ebb)
    n_pad = 10240
    parts_flat = _sc_scatter(col, edge_attr_new.reshape(-1), n_pad, ed)
    parts = parts_flat.reshape(NUM_TILES, n_pad, ed)
    aggr = _sum_partials(parts)
    node_attr_new = _node_mlp(node_attr, aggr[:n],
                              nw1aT, nw1bT, nb1r, nw2T, nb2r, nw3T, nb3r,
                              ng, nbb)
    return (node_attr_new, edge_attr_new)


# speedup vs baseline: 2.1534x; 2.1534x over previous
import functools

import jax
import jax.numpy as jnp
from jax import lax
from jax.experimental import pallas as pl
from jax.experimental.pallas import tpu as pltpu
from jax.experimental.pallas import tpu_sc as plsc

NUM_TILES = 32
CHUNK = 128


def _precompute(node_attr, wsT, wrT, block=2000):
    n, nd = node_attr.shape
    grid = n // block

    def body(x_ref, ws_ref, wr_ref, ps_ref, pr_ref):
        x = x_ref[...]
        ps_ref[...] = jnp.dot(x, ws_ref[...], preferred_element_type=jnp.float32)
        pr_ref[...] = jnp.dot(x, wr_ref[...], preferred_element_type=jnp.float32)

    return pl.pallas_call(
        body,
        grid=(grid,),
        in_specs=[
            pl.BlockSpec((block, nd), lambda i: (i, 0)),
            pl.BlockSpec(wsT.shape, lambda i: (0, 0)),
            pl.BlockSpec(wrT.shape, lambda i: (0, 0)),
        ],
        out_specs=[
            pl.BlockSpec((block, wsT.shape[1]), lambda i: (i, 0)),
            pl.BlockSpec((block, wrT.shape[1]), lambda i: (i, 0)),
        ],
        out_shape=[
            jax.ShapeDtypeStruct((n, wsT.shape[1]), jnp.float32),
            jax.ShapeDtypeStruct((n, wrT.shape[1]), jnp.float32),
        ],
    )(node_attr, wsT, wrT)


def _sc_gather(ps, pr, row, col):
    e = row.shape[0]
    d = ps.shape[1]
    per_tile = e // NUM_TILES
    nfull = per_tile // CHUNK
    tail = per_tile - nfull * CHUNK

    npairs = nfull // 2
    odd = nfull - 2 * npairs

    mesh = plsc.VectorSubcoreMesh(core_axis_name="c", subcore_axis_name="s")

    @functools.partial(
        pl.kernel,
        mesh=mesh,
        out_type=(
            jax.ShapeDtypeStruct((e, d), jnp.float32),
            jax.ShapeDtypeStruct((e, d), jnp.float32),
        ),
        scratch_types=[
            pltpu.VMEM((CHUNK,), jnp.int32),
            pltpu.VMEM((CHUNK,), jnp.int32),
            pltpu.VMEM((CHUNK, d), jnp.float32),
            pltpu.VMEM((CHUNK, d), jnp.float32),
            pltpu.VMEM((CHUNK,), jnp.int32),
            pltpu.VMEM((CHUNK,), jnp.int32),
            pltpu.VMEM((CHUNK, d), jnp.float32),
            pltpu.VMEM((CHUNK, d), jnp.float32),
            pltpu.VMEM((tail,), jnp.int32),
            pltpu.VMEM((tail,), jnp.int32),
            pltpu.VMEM((tail, d), jnp.float32),
            pltpu.VMEM((tail, d), jnp.float32),
            pltpu.SemaphoreType.DMA,
            pltpu.SemaphoreType.DMA,
        ],
    )
    def k(ps_hbm, pr_hbm, row_hbm, col_hbm, s1_hbm, s2_hbm,
          i1, i2, r1, r2, j1, j2, q1, q2, ti1, ti2, tr1, tr2, m1, m2):
        wid = lax.axis_index("c") * 16 + lax.axis_index("s")
        tbase = wid * per_tile

        def body(t, carry):
            baseA = tbase + (2 * t) * CHUNK
            baseB = baseA + CHUNK
            pltpu.sync_copy(row_hbm.at[pl.ds(baseA, CHUNK)], i1)
            pltpu.sync_copy(col_hbm.at[pl.ds(baseA, CHUNK)], i2)
            cpA1 = pltpu.async_copy(ps_hbm.at[i1], r1, m1)
            cpA2 = pltpu.async_copy(pr_hbm.at[i2], r2, m1)
            pltpu.sync_copy(row_hbm.at[pl.ds(baseB, CHUNK)], j1)
            pltpu.sync_copy(col_hbm.at[pl.ds(baseB, CHUNK)], j2)
            cpB1 = pltpu.async_copy(ps_hbm.at[j1], q1, m2)
            cpB2 = pltpu.async_copy(pr_hbm.at[j2], q2, m2)
            cpA1.wait()
            cpA2.wait()
            pltpu.sync_copy(r1, s1_hbm.at[pl.ds(baseA, CHUNK)])
            pltpu.sync_copy(r2, s2_hbm.at[pl.ds(baseA, CHUNK)])
            cpB1.wait()
            cpB2.wait()
            pltpu.sync_copy(q1, s1_hbm.at[pl.ds(baseB, CHUNK)])
            pltpu.sync_copy(q2, s2_hbm.at[pl.ds(baseB, CHUNK)])
            return carry

        lax.fori_loop(0, npairs, body, 0)

        if odd:
            base = tbase + 2 * npairs * CHUNK
            pltpu.sync_copy(row_hbm.at[pl.ds(base, CHUNK)], i1)
            pltpu.sync_copy(col_hbm.at[pl.ds(base, CHUNK)], i2)
            cp1 = pltpu.async_copy(ps_hbm.at[i1], r1, m1)
            cp2 = pltpu.async_copy(pr_hbm.at[i2], r2, m2)
            cp1.wait()
            cp2.wait()
            pltpu.sync_copy(r1, s1_hbm.at[pl.ds(base, CHUNK)])
            pltpu.sync_copy(r2, s2_hbm.at[pl.ds(base, CHUNK)])

        if tail:
            base = tbase + nfull * CHUNK
            pltpu.sync_copy(row_hbm.at[pl.ds(base, tail)], ti1)
            pltpu.sync_copy(col_hbm.at[pl.ds(base, tail)], ti2)
            cp1 = pltpu.async_copy(ps_hbm.at[ti1], tr1, m1)
            cp2 = pltpu.async_copy(pr_hbm.at[ti2], tr2, m2)
            cp1.wait()
            cp2.wait()
            pltpu.sync_copy(tr1, s1_hbm.at[pl.ds(base, tail)])
            pltpu.sync_copy(tr2, s2_hbm.at[pl.ds(base, tail)])

    return k(ps, pr, row, col)


def _edge_mlp(s1, s2, ea, weT, b1, w2T, b2, w3T, b3, g, bb, block=2560):
    e, h = s1.shape
    ed = ea.shape[1]
    grid = e // block

    def body(s1_ref, s2_ref, ea_ref, we_ref, b1_ref, w2_ref, b2_ref,
             w3_ref, b3_ref, g_ref, bb_ref, out_ref):
        ea_blk = ea_ref[...]
        x = (jnp.dot(ea_blk, we_ref[...], preferred_element_type=jnp.float32)
             + s1_ref[...] + s2_ref[...] + b1_ref[...])
        h1 = jnp.maximum(x, 0.0)
        h2 = jnp.maximum(
            jnp.dot(h1, w2_ref[...], preferred_element_type=jnp.float32)
            + b2_ref[...], 0.0)
        u = jnp.dot(h2, w3_ref[...], preferred_element_type=jnp.float32) + b3_ref[...]
        m = jnp.mean(u, axis=-1, keepdims=True)
        c = u - m
        v = jnp.mean(c * c, axis=-1, keepdims=True)
        ln = c * lax.rsqrt(v + 1e-5) * g_ref[...] + bb_ref[...]
        out_ref[...] = ea_blk + ln

    return pl.pallas_call(
        body,
        grid=(grid,),
        in_specs=[
            pl.BlockSpec((block, h), lambda i: (i, 0)),
            pl.BlockSpec((block, h), lambda i: (i, 0)),
            pl.BlockSpec((block, ed), lambda i: (i, 0)),
            pl.BlockSpec(weT.shape, lambda i: (0, 0)),
            pl.BlockSpec(b1.shape, lambda i: (0, 0)),
            pl.BlockSpec(w2T.shape, lambda i: (0, 0)),
            pl.BlockSpec(b2.shape, lambda i: (0, 0)),
            pl.BlockSpec(w3T.shape, lambda i: (0, 0)),
            pl.BlockSpec(b3.shape, lambda i: (0, 0)),
            pl.BlockSpec(g.shape, lambda i: (0, 0)),
            pl.BlockSpec(bb.shape, lambda i: (0, 0)),
        ],
        out_specs=pl.BlockSpec((block, ed), lambda i: (i, 0)),
        out_shape=jax.ShapeDtypeStruct((e, ed), jnp.float32),
    )(s1, s2, ea, weT, b1, w2T, b2, w3T, b3, g, bb)


def _sc_scatter(col, edges_flat, n_pad, d):
    e = edges_flat.shape[0] // d
    cs = 256
    total_chunks = e // cs
    base_chunks = total_chunks // NUM_TILES
    rem_chunks = total_chunks % NUM_TILES
    half = n_pad // 2
    acc_w = half * d
    acc_pad = acc_w + d
    lanes = 16

    mesh = plsc.VectorSubcoreMesh(core_axis_name="c", subcore_axis_name="s")

    @functools.partial(
        pl.kernel,
        mesh=mesh,
        out_type=jax.ShapeDtypeStruct((NUM_TILES * 2 * acc_w,), jnp.float32),
        scratch_types=[
            pltpu.VMEM((cs + 16,), jnp.int32),
            pltpu.VMEM((cs * 16,), jnp.float32),
            pltpu.VMEM((acc_pad,), jnp.float32),
        ],
    )
    def k(col_hbm, edges_hbm, out_hbm, vcol, rows, acc):
        cid = lax.axis_index("c")
        sid = lax.axis_index("s")
        wid = cid * 16 + sid
        my_chunks = base_chunks + jnp.where(wid < rem_chunks, 1, 0)
        start_chunk = wid * base_chunks + jnp.minimum(wid, rem_chunks)
        zvec = jnp.zeros((lanes,), jnp.float32)

        def do_pass(p):
            lo = p * half

            def zbody(i, carry):
                acc[pl.ds(i * lanes, lanes)] = zvec
                return carry
            lax.fori_loop(0, acc_pad // lanes, zbody, 0, unroll=8)

            def body(t, carry):
                base = (start_chunk + t) * cs
                pltpu.sync_copy(col_hbm.at[pl.ds(base, cs)],
                                vcol.at[pl.ds(0, cs)])
                pltpu.sync_copy(edges_hbm.at[pl.ds(base * d, cs * d)],
                                rows)

                def ebody(e2, carry2):
                    loc = vcol[pl.ds(e2, lanes)][0] - lo
                    inr = (loc >= 0) & (loc < half)
                    off = jnp.where(inr, loc, half) * d
                    acc[pl.ds(off, lanes)] = (
                        acc[pl.ds(off, lanes)] + rows[pl.ds(e2 * d, lanes)])
                    return carry2
                lax.fori_loop(0, cs, ebody, 0, unroll=8)
                return carry
            lax.fori_loop(0, my_chunks, body, 0)

            pltpu.sync_copy(
                acc.at[pl.ds(0, acc_w)],
                out_hbm.at[pl.ds((wid * 2 + p) * acc_w, acc_w)])

        do_pass(0)
        do_pass(1)

    return k(col, edges_flat)


def _sum_partials(parts, block=2048):
    nparts, n_pad, ed = parts.shape
    grid_n = n_pad // block

    def body(p_ref, out_ref):
        k = pl.program_id(1)

        @pl.when(k == 0)
        def _():
            out_ref[...] = jnp.zeros_like(out_ref)

        out_ref[...] += p_ref[0]

    return pl.pallas_call(
        body,
        grid=(grid_n, nparts),
        in_specs=[pl.BlockSpec((1, block, ed), lambda i, k: (k, i, 0))],
        out_specs=pl.BlockSpec((block, ed), lambda i, k: (i, 0)),
        out_shape=jax.ShapeDtypeStruct((n_pad, ed), jnp.float32),
    )(parts)


def _node_mlp(na, aggr_in, w1aT, w1bT, b1, w2T, b2, w3T, b3, g, bb, block=2000):
    n, nd = na.shape
    ed = aggr_in.shape[1]
    grid = n // block

    def body(na_ref, aggr_ref, w1a_ref, w1b_ref, b1_ref, w2_ref,
             b2_ref, w3_ref, b3_ref, g_ref, bb_ref, out_ref):
        na_blk = na_ref[...]
        aggr = aggr_ref[...]
        x = (jnp.dot(na_blk, w1a_ref[...], preferred_element_type=jnp.float32)
             + jnp.dot(aggr, w1b_ref[...], preferred_element_type=jnp.float32)
             + b1_ref[...])
        h1 = jnp.maximum(x, 0.0)
        h2 = jnp.maximum(
            jnp.dot(h1, w2_ref[...], preferred_element_type=jnp.float32)
            + b2_ref[...], 0.0)
        u = jnp.dot(h2, w3_ref[...], preferred_element_type=jnp.float32) + b3_ref[...]
        m = jnp.mean(u, axis=-1, keepdims=True)
        c = u - m
        v = jnp.mean(c * c, axis=-1, keepdims=True)
        ln = c * lax.rsqrt(v + 1e-5) * g_ref[...] + bb_ref[...]
        out_ref[...] = na_blk + ln

    return pl.pallas_call(
        body,
        grid=(grid,),
        in_specs=[
            pl.BlockSpec((block, nd), lambda i: (i, 0)),
            pl.BlockSpec((block, ed), lambda i: (i, 0)),
            pl.BlockSpec(w1aT.shape, lambda i: (0, 0)),
            pl.BlockSpec(w1bT.shape, lambda i: (0, 0)),
            pl.BlockSpec(b1.shape, lambda i: (0, 0)),
            pl.BlockSpec(w2T.shape, lambda i: (0, 0)),
            pl.BlockSpec(b2.shape, lambda i: (0, 0)),
            pl.BlockSpec(w3T.shape, lambda i: (0, 0)),
            pl.BlockSpec(b3.shape, lambda i: (0, 0)),
            pl.BlockSpec(g.shape, lambda i: (0, 0)),
            pl.BlockSpec(bb.shape, lambda i: (0, 0)),
        ],
        out_specs=pl.BlockSpec((block, nd), lambda i: (i, 0)),
        out_shape=jax.ShapeDtypeStruct((n, nd), jnp.float32),
    )(na, aggr_in, w1aT, w1bT, b1, w2T, b2, w3T, b3, g, bb)


def kernel(node_attr, edge_attr, edge_index,
           eW1, eb1, eW2, eb2, eW3, eb3, eLNg, eLNb,
           nW1, nb1, nW2, nb2, nW3, nb3, nLNg, nLNb):
    n, nd = node_attr.shape
    e, ed = edge_attr.shape

    row = edge_index[0]
    col = edge_index[1]

    weT = eW1[:, :ed].T
    wsT = eW1[:, ed:ed + nd].T
    wrT = eW1[:, ed + nd:].T
    ew2T = eW2.T
    ew3T = eW3.T
    eb1r = eb1.reshape(1, -1)
    eb2r = eb2.reshape(1, -1)
    eb3r = eb3.reshape(1, -1)
    eg = eLNg.reshape(1, -1)
    ebb = eLNb.reshape(1, -1)

    nw1aT = nW1[:, :nd].T
    nw1bT = nW1[:, nd:].T
    nw2T = nW2.T
    nw3T = nW3.T
    nb1r = nb1.reshape(1, -1)
    nb2r = nb2.reshape(1, -1)
    nb3r = nb3.reshape(1, -1)
    ng = nLNg.reshape(1, -1)
    nbb = nLNb.reshape(1, -1)

    ps, pr = _precompute(node_attr, wsT, wrT)
    s1, s2 = _sc_gather(ps, pr, row, col)
    edge_attr_new = _edge_mlp(s1, s2, edge_attr, weT, eb1r,
                              ew2T, eb2r, ew3T, eb3r, eg, ebb)
    n_pad = 10240
    parts_flat = _sc_scatter(col, edge_attr_new.reshape(-1), n_pad, ed)
    parts = parts_flat.reshape(NUM_TILES, n_pad, ed)
    aggr = _sum_partials(parts)
    node_attr_new = _node_mlp(node_attr, aggr[:n],
                              nw1aT, nw1bT, nb1r, nw2T, nb2r, nw3T, nb3r,
                              ng, nbb)
    return (node_attr_new, edge_attr_new)


# speedup vs baseline: 2.2677x; 1.0531x over previous
import functools

import jax
import jax.numpy as jnp
from jax import lax
from jax.experimental import pallas as pl
from jax.experimental.pallas import tpu as pltpu
from jax.experimental.pallas import tpu_sc as plsc

NUM_TILES = 32
CHUNK = 128


def _precompute(node_attr, wsT, wrT, block=2000):
    n, nd = node_attr.shape
    grid = n // block

    def body(x_ref, ws_ref, wr_ref, ps_ref, pr_ref):
        x = x_ref[...]
        ps_ref[...] = jnp.dot(x, ws_ref[...], preferred_element_type=jnp.float32)
        pr_ref[...] = jnp.dot(x, wr_ref[...], preferred_element_type=jnp.float32)

    return pl.pallas_call(
        body,
        grid=(grid,),
        in_specs=[
            pl.BlockSpec((block, nd), lambda i: (i, 0)),
            pl.BlockSpec(wsT.shape, lambda i: (0, 0)),
            pl.BlockSpec(wrT.shape, lambda i: (0, 0)),
        ],
        out_specs=[
            pl.BlockSpec((block, wsT.shape[1]), lambda i: (i, 0)),
            pl.BlockSpec((block, wrT.shape[1]), lambda i: (i, 0)),
        ],
        out_shape=[
            jax.ShapeDtypeStruct((n, wsT.shape[1]), jnp.float32),
            jax.ShapeDtypeStruct((n, wrT.shape[1]), jnp.float32),
        ],
    )(node_attr, wsT, wrT)


def _sc_gather(ps, pr, row, col):
    e = row.shape[0]
    d = ps.shape[1]
    per_tile = e // NUM_TILES
    nfull = per_tile // CHUNK
    tail = per_tile - nfull * CHUNK

    npairs = nfull // 2
    odd = nfull - 2 * npairs

    mesh = plsc.VectorSubcoreMesh(core_axis_name="c", subcore_axis_name="s")

    @functools.partial(
        pl.kernel,
        mesh=mesh,
        out_type=(
            jax.ShapeDtypeStruct((e, d), jnp.float32),
            jax.ShapeDtypeStruct((e, d), jnp.float32),
        ),
        scratch_types=[
            pltpu.VMEM((CHUNK,), jnp.int32),
            pltpu.VMEM((CHUNK,), jnp.int32),
            pltpu.VMEM((CHUNK, d), jnp.float32),
            pltpu.VMEM((CHUNK, d), jnp.float32),
            pltpu.VMEM((CHUNK,), jnp.int32),
            pltpu.VMEM((CHUNK,), jnp.int32),
            pltpu.VMEM((CHUNK, d), jnp.float32),
            pltpu.VMEM((CHUNK, d), jnp.float32),
            pltpu.VMEM((tail,), jnp.int32),
            pltpu.VMEM((tail,), jnp.int32),
            pltpu.VMEM((tail, d), jnp.float32),
            pltpu.VMEM((tail, d), jnp.float32),
            pltpu.SemaphoreType.DMA,
            pltpu.SemaphoreType.DMA,
        ],
    )
    def k(ps_hbm, pr_hbm, row_hbm, col_hbm, s1_hbm, s2_hbm,
          i1, i2, r1, r2, j1, j2, q1, q2, ti1, ti2, tr1, tr2, m1, m2):
        wid = lax.axis_index("c") * 16 + lax.axis_index("s")
        tbase = wid * per_tile

        def body(t, carry):
            baseA = tbase + (2 * t) * CHUNK
            baseB = baseA + CHUNK
            pltpu.sync_copy(row_hbm.at[pl.ds(baseA, CHUNK)], i1)
            pltpu.sync_copy(col_hbm.at[pl.ds(baseA, CHUNK)], i2)
            cpA1 = pltpu.async_copy(ps_hbm.at[i1], r1, m1)
            cpA2 = pltpu.async_copy(pr_hbm.at[i2], r2, m1)
            pltpu.sync_copy(row_hbm.at[pl.ds(baseB, CHUNK)], j1)
            pltpu.sync_copy(col_hbm.at[pl.ds(baseB, CHUNK)], j2)
            cpB1 = pltpu.async_copy(ps_hbm.at[j1], q1, m2)
            cpB2 = pltpu.async_copy(pr_hbm.at[j2], q2, m2)
            cpA1.wait()
            cpA2.wait()
            pltpu.sync_copy(r1, s1_hbm.at[pl.ds(baseA, CHUNK)])
            pltpu.sync_copy(r2, s2_hbm.at[pl.ds(baseA, CHUNK)])
            cpB1.wait()
            cpB2.wait()
            pltpu.sync_copy(q1, s1_hbm.at[pl.ds(baseB, CHUNK)])
            pltpu.sync_copy(q2, s2_hbm.at[pl.ds(baseB, CHUNK)])
            return carry

        lax.fori_loop(0, npairs, body, 0)

        if odd:
            base = tbase + 2 * npairs * CHUNK
            pltpu.sync_copy(row_hbm.at[pl.ds(base, CHUNK)], i1)
            pltpu.sync_copy(col_hbm.at[pl.ds(base, CHUNK)], i2)
            cp1 = pltpu.async_copy(ps_hbm.at[i1], r1, m1)
            cp2 = pltpu.async_copy(pr_hbm.at[i2], r2, m2)
            cp1.wait()
            cp2.wait()
            pltpu.sync_copy(r1, s1_hbm.at[pl.ds(base, CHUNK)])
            pltpu.sync_copy(r2, s2_hbm.at[pl.ds(base, CHUNK)])

        if tail:
            base = tbase + nfull * CHUNK
            pltpu.sync_copy(row_hbm.at[pl.ds(base, tail)], ti1)
            pltpu.sync_copy(col_hbm.at[pl.ds(base, tail)], ti2)
            cp1 = pltpu.async_copy(ps_hbm.at[ti1], tr1, m1)
            cp2 = pltpu.async_copy(pr_hbm.at[ti2], tr2, m2)
            cp1.wait()
            cp2.wait()
            pltpu.sync_copy(tr1, s1_hbm.at[pl.ds(base, tail)])
            pltpu.sync_copy(tr2, s2_hbm.at[pl.ds(base, tail)])

    return k(ps, pr, row, col)


def _edge_mlp(s1, s2, ea, weT, b1, w2T, b2, w3T, b3, g, bb, block=2560):
    e, h = s1.shape
    ed = ea.shape[1]
    grid = e // block

    def body(s1_ref, s2_ref, ea_ref, we_ref, b1_ref, w2_ref, b2_ref,
             w3_ref, b3_ref, g_ref, bb_ref, out_ref):
        ea_blk = ea_ref[...]
        x = (jnp.dot(ea_blk, we_ref[...], preferred_element_type=jnp.float32)
             + s1_ref[...] + s2_ref[...] + b1_ref[...])
        h1 = jnp.maximum(x, 0.0)
        h2 = jnp.maximum(
            jnp.dot(h1, w2_ref[...], preferred_element_type=jnp.float32)
            + b2_ref[...], 0.0)
        u = jnp.dot(h2, w3_ref[...], preferred_element_type=jnp.float32) + b3_ref[...]
        m = jnp.mean(u, axis=-1, keepdims=True)
        c = u - m
        v = jnp.mean(c * c, axis=-1, keepdims=True)
        ln = c * lax.rsqrt(v + 1e-5) * g_ref[...] + bb_ref[...]
        out_ref[...] = ea_blk + ln

    return pl.pallas_call(
        body,
        grid=(grid,),
        in_specs=[
            pl.BlockSpec((block, h), lambda i: (i, 0)),
            pl.BlockSpec((block, h), lambda i: (i, 0)),
            pl.BlockSpec((block, ed), lambda i: (i, 0)),
            pl.BlockSpec(weT.shape, lambda i: (0, 0)),
            pl.BlockSpec(b1.shape, lambda i: (0, 0)),
            pl.BlockSpec(w2T.shape, lambda i: (0, 0)),
            pl.BlockSpec(b2.shape, lambda i: (0, 0)),
            pl.BlockSpec(w3T.shape, lambda i: (0, 0)),
            pl.BlockSpec(b3.shape, lambda i: (0, 0)),
            pl.BlockSpec(g.shape, lambda i: (0, 0)),
            pl.BlockSpec(bb.shape, lambda i: (0, 0)),
        ],
        out_specs=pl.BlockSpec((block, ed), lambda i: (i, 0)),
        out_shape=jax.ShapeDtypeStruct((e, ed), jnp.float32),
    )(s1, s2, ea, weT, b1, w2T, b2, w3T, b3, g, bb)


def _sc_scatter(col, edges_flat, n_pad, d):
    e = edges_flat.shape[0] // d
    cs = 256
    total_chunks = e // cs
    base_chunks = total_chunks // NUM_TILES
    rem_chunks = total_chunks % NUM_TILES
    half = n_pad // 2
    acc_w = half * d
    acc_pad = acc_w + d
    lanes = 16

    mesh = plsc.VectorSubcoreMesh(core_axis_name="c", subcore_axis_name="s")

    @functools.partial(
        pl.kernel,
        mesh=mesh,
        out_type=jax.ShapeDtypeStruct((NUM_TILES * 2 * acc_w,), jnp.float32),
        scratch_types=[
            pltpu.VMEM((cs + 16,), jnp.int32),
            pltpu.VMEM((cs * 16,), jnp.float32),
            pltpu.VMEM((cs + 16,), jnp.int32),
            pltpu.VMEM((cs * 16,), jnp.float32),
            pltpu.VMEM((acc_pad,), jnp.float32),
            pltpu.SemaphoreType.DMA,
            pltpu.SemaphoreType.DMA,
        ],
    )
    def k(col_hbm, edges_hbm, out_hbm, vcol, rows, vcol2, rows2, acc,
          ma, mb):
        cid = lax.axis_index("c")
        sid = lax.axis_index("s")
        wid = cid * 16 + sid
        my_chunks = base_chunks + jnp.where(wid < rem_chunks, 1, 0)
        start_chunk = wid * base_chunks + jnp.minimum(wid, rem_chunks)
        zvec = jnp.zeros((lanes,), jnp.float32)

        def do_pass(p):
            lo = p * half

            def zbody(i, carry):
                acc[pl.ds(i * lanes, lanes)] = zvec
                return carry
            lax.fori_loop(0, acc_pad // lanes, zbody, 0, unroll=8)

            def accumulate(cbuf, rbuf):
                def ebody(e2, carry2):
                    loc = cbuf[pl.ds(e2, lanes)][0] - lo
                    inr = (loc >= 0) & (loc < half)
                    off = jnp.where(inr, loc, half) * d
                    acc[pl.ds(off, lanes)] = (
                        acc[pl.ds(off, lanes)] + rbuf[pl.ds(e2 * d, lanes)])
                    return carry2
                lax.fori_loop(0, cs, ebody, 0, unroll=8)

            def stage(base, cbuf, rbuf, sem):
                c1 = pltpu.async_copy(col_hbm.at[pl.ds(base, cs)],
                                      cbuf.at[pl.ds(0, cs)], sem)
                c2 = pltpu.async_copy(edges_hbm.at[pl.ds(base * d, cs * d)],
                                      rbuf, sem)
                return c1, c2

            npairs_t = my_chunks // 2

            def body(t, carry):
                baseA = (start_chunk + 2 * t) * cs
                baseB = baseA + cs
                a1, a2 = stage(baseA, vcol, rows, ma)
                b1, b2 = stage(baseB, vcol2, rows2, mb)
                a1.wait()
                a2.wait()
                accumulate(vcol, rows)
                b1.wait()
                b2.wait()
                accumulate(vcol2, rows2)
                return carry
            lax.fori_loop(0, npairs_t, body, 0)

            @pl.when(my_chunks % 2 == 1)
            def _():
                base = (start_chunk + 2 * npairs_t) * cs
                a1, a2 = stage(base, vcol, rows, ma)
                a1.wait()
                a2.wait()
                accumulate(vcol, rows)

            pltpu.sync_copy(
                acc.at[pl.ds(0, acc_w)],
                out_hbm.at[pl.ds((wid * 2 + p) * acc_w, acc_w)])

        do_pass(0)
        do_pass(1)

    return k(col, edges_flat)


def _sum_partials(parts, block=2048):
    nparts, n_pad, ed = parts.shape
    grid_n = n_pad // block

    def body(p_ref, out_ref):
        k = pl.program_id(1)

        @pl.when(k == 0)
        def _():
            out_ref[...] = jnp.zeros_like(out_ref)

        out_ref[...] += p_ref[0]

    return pl.pallas_call(
        body,
        grid=(grid_n, nparts),
        in_specs=[pl.BlockSpec((1, block, ed), lambda i, k: (k, i, 0))],
        out_specs=pl.BlockSpec((block, ed), lambda i, k: (i, 0)),
        out_shape=jax.ShapeDtypeStruct((n_pad, ed), jnp.float32),
    )(parts)


def _node_mlp(na, aggr_in, w1aT, w1bT, b1, w2T, b2, w3T, b3, g, bb, block=2000):
    n, nd = na.shape
    ed = aggr_in.shape[1]
    grid = n // block

    def body(na_ref, aggr_ref, w1a_ref, w1b_ref, b1_ref, w2_ref,
             b2_ref, w3_ref, b3_ref, g_ref, bb_ref, out_ref):
        na_blk = na_ref[...]
        aggr = aggr_ref[...]
        x = (jnp.dot(na_blk, w1a_ref[...], preferred_element_type=jnp.float32)
             + jnp.dot(aggr, w1b_ref[...], preferred_element_type=jnp.float32)
             + b1_ref[...])
        h1 = jnp.maximum(x, 0.0)
        h2 = jnp.maximum(
            jnp.dot(h1, w2_ref[...], preferred_element_type=jnp.float32)
            + b2_ref[...], 0.0)
        u = jnp.dot(h2, w3_ref[...], preferred_element_type=jnp.float32) + b3_ref[...]
        m = jnp.mean(u, axis=-1, keepdims=True)
        c = u - m
        v = jnp.mean(c * c, axis=-1, keepdims=True)
        ln = c * lax.rsqrt(v + 1e-5) * g_ref[...] + bb_ref[...]
        out_ref[...] = na_blk + ln

    return pl.pallas_call(
        body,
        grid=(grid,),
        in_specs=[
            pl.BlockSpec((block, nd), lambda i: (i, 0)),
            pl.BlockSpec((block, ed), lambda i: (i, 0)),
            pl.BlockSpec(w1aT.shape, lambda i: (0, 0)),
            pl.BlockSpec(w1bT.shape, lambda i: (0, 0)),
            pl.BlockSpec(b1.shape, lambda i: (0, 0)),
            pl.BlockSpec(w2T.shape, lambda i: (0, 0)),
            pl.BlockSpec(b2.shape, lambda i: (0, 0)),
            pl.BlockSpec(w3T.shape, lambda i: (0, 0)),
            pl.BlockSpec(b3.shape, lambda i: (0, 0)),
            pl.BlockSpec(g.shape, lambda i: (0, 0)),
            pl.BlockSpec(bb.shape, lambda i: (0, 0)),
        ],
        out_specs=pl.BlockSpec((block, nd), lambda i: (i, 0)),
        out_shape=jax.ShapeDtypeStruct((n, nd), jnp.float32),
    )(na, aggr_in, w1aT, w1bT, b1, w2T, b2, w3T, b3, g, bb)


def kernel(node_attr, edge_attr, edge_index,
           eW1, eb1, eW2, eb2, eW3, eb3, eLNg, eLNb,
           nW1, nb1, nW2, nb2, nW3, nb3, nLNg, nLNb):
    n, nd = node_attr.shape
    e, ed = edge_attr.shape

    row = edge_index[0]
    col = edge_index[1]

    weT = eW1[:, :ed].T
    wsT = eW1[:, ed:ed + nd].T
    wrT = eW1[:, ed + nd:].T
    ew2T = eW2.T
    ew3T = eW3.T
    eb1r = eb1.reshape(1, -1)
    eb2r = eb2.reshape(1, -1)
    eb3r = eb3.reshape(1, -1)
    eg = eLNg.reshape(1, -1)
    ebb = eLNb.reshape(1, -1)

    nw1aT = nW1[:, :nd].T
    nw1bT = nW1[:, nd:].T
    nw2T = nW2.T
    nw3T = nW3.T
    nb1r = nb1.reshape(1, -1)
    nb2r = nb2.reshape(1, -1)
    nb3r = nb3.reshape(1, -1)
    ng = nLNg.reshape(1, -1)
    nbb = nLNb.reshape(1, -1)

    ps, pr = _precompute(node_attr, wsT, wrT)
    s1, s2 = _sc_gather(ps, pr, row, col)
    edge_attr_new = _edge_mlp(s1, s2, edge_attr, weT, eb1r,
                              ew2T, eb2r, ew3T, eb3r, eg, ebb)
    n_pad = 10240
    parts_flat = _sc_scatter(col, edge_attr_new.reshape(-1), n_pad, ed)
    parts = parts_flat.reshape(NUM_TILES, n_pad, ed)
    aggr = _sum_partials(parts)
    node_attr_new = _node_mlp(node_attr, aggr[:n],
                              nw1aT, nw1bT, nb1r, nw2T, nb2r, nw3T, nb3r,
                              ng, nbb)
    return (node_attr_new, edge_attr_new)


# speedup vs baseline: 2.4079x; 1.0618x over previous
import functools

import jax
import jax.numpy as jnp
from jax import lax
from jax.experimental import pallas as pl
from jax.experimental.pallas import tpu as pltpu
from jax.experimental.pallas import tpu_sc as plsc

NUM_TILES = 32
CHUNK = 128


def _precompute(node_attr, wsT, wrT, block=2000):
    n, nd = node_attr.shape
    grid = n // block

    def body(x_ref, ws_ref, wr_ref, ps_ref, pr_ref):
        x = x_ref[...]
        ps_ref[...] = jnp.dot(x, ws_ref[...], preferred_element_type=jnp.float32)
        pr_ref[...] = jnp.dot(x, wr_ref[...], preferred_element_type=jnp.float32)

    return pl.pallas_call(
        body,
        grid=(grid,),
        in_specs=[
            pl.BlockSpec((block, nd), lambda i: (i, 0)),
            pl.BlockSpec(wsT.shape, lambda i: (0, 0)),
            pl.BlockSpec(wrT.shape, lambda i: (0, 0)),
        ],
        out_specs=[
            pl.BlockSpec((block, wsT.shape[1]), lambda i: (i, 0)),
            pl.BlockSpec((block, wrT.shape[1]), lambda i: (i, 0)),
        ],
        out_shape=[
            jax.ShapeDtypeStruct((n, wsT.shape[1]), jnp.float32),
            jax.ShapeDtypeStruct((n, wrT.shape[1]), jnp.float32),
        ],
    )(node_attr, wsT, wrT)


def _sc_gather(ps, pr, row, col):
    e = row.shape[0]
    d = ps.shape[1]
    per_tile = e // NUM_TILES
    nfull = per_tile // CHUNK
    tail = per_tile - nfull * CHUNK

    npairs = nfull // 2
    odd = nfull - 2 * npairs

    mesh = plsc.VectorSubcoreMesh(core_axis_name="c", subcore_axis_name="s")

    @functools.partial(
        pl.kernel,
        mesh=mesh,
        out_type=(
            jax.ShapeDtypeStruct((e, d), jnp.float32),
            jax.ShapeDtypeStruct((e, d), jnp.float32),
        ),
        scratch_types=[
            pltpu.VMEM((CHUNK,), jnp.int32),
            pltpu.VMEM((CHUNK,), jnp.int32),
            pltpu.VMEM((CHUNK, d), jnp.float32),
            pltpu.VMEM((CHUNK, d), jnp.float32),
            pltpu.VMEM((CHUNK,), jnp.int32),
            pltpu.VMEM((CHUNK,), jnp.int32),
            pltpu.VMEM((CHUNK, d), jnp.float32),
            pltpu.VMEM((CHUNK, d), jnp.float32),
            pltpu.VMEM((tail,), jnp.int32),
            pltpu.VMEM((tail,), jnp.int32),
            pltpu.VMEM((tail, d), jnp.float32),
            pltpu.VMEM((tail, d), jnp.float32),
            pltpu.SemaphoreType.DMA,
            pltpu.SemaphoreType.DMA,
        ],
    )
    def k(ps_hbm, pr_hbm, row_hbm, col_hbm, s1_hbm, s2_hbm,
          i1, i2, r1, r2, j1, j2, q1, q2, ti1, ti2, tr1, tr2, m1, m2):
        wid = lax.axis_index("c") * 16 + lax.axis_index("s")
        tbase = wid * per_tile

        def body(t, carry):
            baseA = tbase + (2 * t) * CHUNK
            baseB = baseA + CHUNK
            pltpu.sync_copy(row_hbm.at[pl.ds(baseA, CHUNK)], i1)
            pltpu.sync_copy(col_hbm.at[pl.ds(baseA, CHUNK)], i2)
            cpA1 = pltpu.async_copy(ps_hbm.at[i1], r1, m1)
            cpA2 = pltpu.async_copy(pr_hbm.at[i2], r2, m1)
            pltpu.sync_copy(row_hbm.at[pl.ds(baseB, CHUNK)], j1)
            pltpu.sync_copy(col_hbm.at[pl.ds(baseB, CHUNK)], j2)
            cpB1 = pltpu.async_copy(ps_hbm.at[j1], q1, m2)
            cpB2 = pltpu.async_copy(pr_hbm.at[j2], q2, m2)
            cpA1.wait()
            cpA2.wait()
            pltpu.sync_copy(r1, s1_hbm.at[pl.ds(baseA, CHUNK)])
            pltpu.sync_copy(r2, s2_hbm.at[pl.ds(baseA, CHUNK)])
            cpB1.wait()
            cpB2.wait()
            pltpu.sync_copy(q1, s1_hbm.at[pl.ds(baseB, CHUNK)])
            pltpu.sync_copy(q2, s2_hbm.at[pl.ds(baseB, CHUNK)])
            return carry

        lax.fori_loop(0, npairs, body, 0)

        if odd:
            base = tbase + 2 * npairs * CHUNK
            pltpu.sync_copy(row_hbm.at[pl.ds(base, CHUNK)], i1)
            pltpu.sync_copy(col_hbm.at[pl.ds(base, CHUNK)], i2)
            cp1 = pltpu.async_copy(ps_hbm.at[i1], r1, m1)
            cp2 = pltpu.async_copy(pr_hbm.at[i2], r2, m2)
            cp1.wait()
            cp2.wait()
            pltpu.sync_copy(r1, s1_hbm.at[pl.ds(base, CHUNK)])
            pltpu.sync_copy(r2, s2_hbm.at[pl.ds(base, CHUNK)])

        if tail:
            base = tbase + nfull * CHUNK
            pltpu.sync_copy(row_hbm.at[pl.ds(base, tail)], ti1)
            pltpu.sync_copy(col_hbm.at[pl.ds(base, tail)], ti2)
            cp1 = pltpu.async_copy(ps_hbm.at[ti1], tr1, m1)
            cp2 = pltpu.async_copy(pr_hbm.at[ti2], tr2, m2)
            cp1.wait()
            cp2.wait()
            pltpu.sync_copy(tr1, s1_hbm.at[pl.ds(base, tail)])
            pltpu.sync_copy(tr2, s2_hbm.at[pl.ds(base, tail)])

    return k(ps, pr, row, col)


def _edge_mlp(s1, s2, ea, weT, b1, w2T, b2, w3T, b3, g, bb, block=2560):
    e, h = s1.shape
    ed = ea.shape[1]
    grid = e // block

    def body(s1_ref, s2_ref, ea_ref, we_ref, b1_ref, w2_ref, b2_ref,
             w3_ref, b3_ref, g_ref, bb_ref, out_ref):
        ea_blk = ea_ref[...]
        x = (jnp.dot(ea_blk, we_ref[...], preferred_element_type=jnp.float32)
             + s1_ref[...] + s2_ref[...] + b1_ref[...])
        h1 = jnp.maximum(x, 0.0)
        h2 = jnp.maximum(
            jnp.dot(h1, w2_ref[...], preferred_element_type=jnp.float32)
            + b2_ref[...], 0.0)
        u = jnp.dot(h2, w3_ref[...], preferred_element_type=jnp.float32) + b3_ref[...]
        m = jnp.mean(u, axis=-1, keepdims=True)
        c = u - m
        v = jnp.mean(c * c, axis=-1, keepdims=True)
        ln = c * lax.rsqrt(v + 1e-5) * g_ref[...] + bb_ref[...]
        out_ref[...] = ea_blk + ln

    return pl.pallas_call(
        body,
        grid=(grid,),
        in_specs=[
            pl.BlockSpec((block, h), lambda i: (i, 0)),
            pl.BlockSpec((block, h), lambda i: (i, 0)),
            pl.BlockSpec((block, ed), lambda i: (i, 0)),
            pl.BlockSpec(weT.shape, lambda i: (0, 0)),
            pl.BlockSpec(b1.shape, lambda i: (0, 0)),
            pl.BlockSpec(w2T.shape, lambda i: (0, 0)),
            pl.BlockSpec(b2.shape, lambda i: (0, 0)),
            pl.BlockSpec(w3T.shape, lambda i: (0, 0)),
            pl.BlockSpec(b3.shape, lambda i: (0, 0)),
            pl.BlockSpec(g.shape, lambda i: (0, 0)),
            pl.BlockSpec(bb.shape, lambda i: (0, 0)),
        ],
        out_specs=pl.BlockSpec((block, ed), lambda i: (i, 0)),
        out_shape=jax.ShapeDtypeStruct((e, ed), jnp.float32),
    )(s1, s2, ea, weT, b1, w2T, b2, w3T, b3, g, bb)


def _sc_scatter(col, edges_flat, n_pad, d):
    e = edges_flat.shape[0] // d
    cs = 256
    total_chunks = e // cs
    base_chunks = total_chunks // NUM_TILES
    rem_chunks = total_chunks % NUM_TILES
    half = n_pad // 2
    acc_w = half * d
    acc_pad = acc_w + d
    lanes = 16

    mesh = plsc.VectorSubcoreMesh(core_axis_name="c", subcore_axis_name="s")

    @functools.partial(
        pl.kernel,
        mesh=mesh,
        out_type=jax.ShapeDtypeStruct((NUM_TILES * 2 * acc_w,), jnp.float32),
        scratch_types=[
            pltpu.VMEM((cs + 16,), jnp.int32),
            pltpu.VMEM((cs * 16,), jnp.float32),
            pltpu.VMEM((cs + 16,), jnp.int32),
            pltpu.VMEM((cs * 16,), jnp.float32),
            pltpu.VMEM((acc_pad,), jnp.float32),
            pltpu.SemaphoreType.DMA,
            pltpu.SemaphoreType.DMA,
        ],
    )
    def k(col_hbm, edges_hbm, out_hbm, vcol, rows, vcol2, rows2, acc,
          ma, mb):
        cid = lax.axis_index("c")
        sid = lax.axis_index("s")
        wid = cid * 16 + sid
        my_chunks = base_chunks + jnp.where(wid < rem_chunks, 1, 0)
        start_chunk = wid * base_chunks + jnp.minimum(wid, rem_chunks)
        zvec = jnp.zeros((lanes,), jnp.float32)

        def do_pass(p):
            lo = p * half

            def zbody(i, carry):
                acc[pl.ds(i * lanes, lanes)] = zvec
                return carry
            lax.fori_loop(0, acc_pad // lanes, zbody, 0, unroll=8)

            def accumulate(cbuf, rbuf):
                def ebody(e2, carry2):
                    loc = cbuf[pl.ds(e2, lanes)][0] - lo
                    inr = (loc >= 0) & (loc < half)
                    off = jnp.where(inr, loc, half) * d
                    acc[pl.ds(off, lanes)] = (
                        acc[pl.ds(off, lanes)] + rbuf[pl.ds(e2 * d, lanes)])
                    return carry2
                lax.fori_loop(0, cs, ebody, 0, unroll=8)

            def stage(base, cbuf, rbuf, sem):
                c1 = pltpu.async_copy(col_hbm.at[pl.ds(base, cs)],
                                      cbuf.at[pl.ds(0, cs)], sem)
                c2 = pltpu.async_copy(edges_hbm.at[pl.ds(base * d, cs * d)],
                                      rbuf, sem)
                return c1, c2

            npairs_t = my_chunks // 2

            def body(t, carry):
                baseA = (start_chunk + 2 * t) * cs
                baseB = baseA + cs
                a1, a2 = stage(baseA, vcol, rows, ma)
                b1, b2 = stage(baseB, vcol2, rows2, mb)
                a1.wait()
                a2.wait()
                accumulate(vcol, rows)
                b1.wait()
                b2.wait()
                accumulate(vcol2, rows2)
                return carry
            lax.fori_loop(0, npairs_t, body, 0)

            @pl.when(my_chunks % 2 == 1)
            def _():
                base = (start_chunk + 2 * npairs_t) * cs
                a1, a2 = stage(base, vcol, rows, ma)
                a1.wait()
                a2.wait()
                accumulate(vcol, rows)

            pltpu.sync_copy(
                acc.at[pl.ds(0, acc_w)],
                out_hbm.at[pl.ds((wid * 2 + p) * acc_w, acc_w)])

        do_pass(0)
        do_pass(1)

    return k(col, edges_flat)


def _sum_partials(parts, block=2048):
    nparts, n_pad, ed = parts.shape
    grid_n = n_pad // block

    def body(p_ref, out_ref):
        k = pl.program_id(1)

        @pl.when(k == 0)
        def _():
            out_ref[...] = jnp.zeros_like(out_ref)

        out_ref[...] += p_ref[0]

    return pl.pallas_call(
        body,
        grid=(grid_n, nparts),
        in_specs=[pl.BlockSpec((1, block, ed), lambda i, k: (k, i, 0))],
        out_specs=pl.BlockSpec((block, ed), lambda i, k: (i, 0)),
        out_shape=jax.ShapeDtypeStruct((n_pad, ed), jnp.float32),
    )(parts)


def _node_mlp(na, parts, w1aT, w1bT, b1, w2T, b2, w3T, b3, g, bb, block=1000):
    n, nd = na.shape
    nparts, n_pad, ed = parts.shape
    grid = n // block

    def body(na_ref, parts_ref, w1a_ref, w1b_ref, b1_ref, w2_ref,
             b2_ref, w3_ref, b3_ref, g_ref, bb_ref, out_ref):
        na_blk = na_ref[...]
        aggr = jnp.sum(parts_ref[...], axis=0)
        x = (jnp.dot(na_blk, w1a_ref[...], preferred_element_type=jnp.float32)
             + jnp.dot(aggr, w1b_ref[...], preferred_element_type=jnp.float32)
             + b1_ref[...])
        h1 = jnp.maximum(x, 0.0)
        h2 = jnp.maximum(
            jnp.dot(h1, w2_ref[...], preferred_element_type=jnp.float32)
            + b2_ref[...], 0.0)
        u = jnp.dot(h2, w3_ref[...], preferred_element_type=jnp.float32) + b3_ref[...]
        m = jnp.mean(u, axis=-1, keepdims=True)
        c = u - m
        v = jnp.mean(c * c, axis=-1, keepdims=True)
        ln = c * lax.rsqrt(v + 1e-5) * g_ref[...] + bb_ref[...]
        out_ref[...] = na_blk + ln

    return pl.pallas_call(
        body,
        grid=(grid,),
        in_specs=[
            pl.BlockSpec((block, nd), lambda i: (i, 0)),
            pl.BlockSpec((nparts, block, ed), lambda i: (0, i, 0)),
            pl.BlockSpec(w1aT.shape, lambda i: (0, 0)),
            pl.BlockSpec(w1bT.shape, lambda i: (0, 0)),
            pl.BlockSpec(b1.shape, lambda i: (0, 0)),
            pl.BlockSpec(w2T.shape, lambda i: (0, 0)),
            pl.BlockSpec(b2.shape, lambda i: (0, 0)),
            pl.BlockSpec(w3T.shape, lambda i: (0, 0)),
            pl.BlockSpec(b3.shape, lambda i: (0, 0)),
            pl.BlockSpec(g.shape, lambda i: (0, 0)),
            pl.BlockSpec(bb.shape, lambda i: (0, 0)),
        ],
        out_specs=pl.BlockSpec((block, nd), lambda i: (i, 0)),
        out_shape=jax.ShapeDtypeStruct((n, nd), jnp.float32),
    )(na, parts, w1aT, w1bT, b1, w2T, b2, w3T, b3, g, bb)


def kernel(node_attr, edge_attr, edge_index,
           eW1, eb1, eW2, eb2, eW3, eb3, eLNg, eLNb,
           nW1, nb1, nW2, nb2, nW3, nb3, nLNg, nLNb):
    n, nd = node_attr.shape
    e, ed = edge_attr.shape

    row = edge_index[0]
    col = edge_index[1]

    weT = eW1[:, :ed].T
    wsT = eW1[:, ed:ed + nd].T
    wrT = eW1[:, ed + nd:].T
    ew2T = eW2.T
    ew3T = eW3.T
    eb1r = eb1.reshape(1, -1)
    eb2r = eb2.reshape(1, -1)
    eb3r = eb3.reshape(1, -1)
    eg = eLNg.reshape(1, -1)
    ebb = eLNb.reshape(1, -1)

    nw1aT = nW1[:, :nd].T
    nw1bT = nW1[:, nd:].T
    nw2T = nW2.T
    nw3T = nW3.T
    nb1r = nb1.reshape(1, -1)
    nb2r = nb2.reshape(1, -1)
    nb3r = nb3.reshape(1, -1)
    ng = nLNg.reshape(1, -1)
    nbb = nLNb.reshape(1, -1)

    ps, pr = _precompute(node_attr, wsT, wrT)
    s1, s2 = _sc_gather(ps, pr, row, col)
    edge_attr_new = _edge_mlp(s1, s2, edge_attr, weT, eb1r,
                              ew2T, eb2r, ew3T, eb3r, eg, ebb)
    n_pad = 10240
    parts_flat = _sc_scatter(col, edge_attr_new.reshape(-1), n_pad, ed)
    parts = parts_flat.reshape(NUM_TILES, n_pad, ed)
    node_attr_new = _node_mlp(node_attr, parts,
                              nw1aT, nw1bT, nb1r, nw2T, nb2r, nw3T, nb3r,
                              ng, nbb)
    return (node_attr_new, edge_attr_new)
